# baseline (speedup 1.0000x reference)
import jax
import jax.numpy as jnp
from jax.experimental import pallas as pl
from jax.experimental.pallas import tpu as pltpu

_WIDTH = 2048
_KPAD = 2176
_NE = 64
_BM = 1024
_K1 = 1024
_K2 = 1024
_K3 = 128


def _router_body(x1_ref, x2_ref, x3_ref, w1_ref, w2_ref, w3_ref, b_ref, o_ref):
    acc = jnp.dot(x1_ref[...], w1_ref[...], preferred_element_type=jnp.float32)
    acc += jnp.dot(x2_ref[...], w2_ref[...], preferred_element_type=jnp.float32)
    acc += jnp.dot(x3_ref[...], w3_ref[...], preferred_element_type=jnp.float32)
    o_ref[...] = acc + b_ref[...]


def kernel(x, A, W, b):
    n = x.shape[0]
    off = (A - _WIDTH).astype(jnp.int32) if hasattr(A, "astype") else jnp.int32(A - _WIDTH)
    wp = jax.lax.dynamic_update_slice(
        jnp.zeros((_KPAD, _NE), jnp.float32), W.T.astype(jnp.float32), (off, 0)
    )
    w1 = wp[:_K1]
    w2 = wp[_K1:_K1 + _K2]
    w3 = wp[_K1 + _K2:]
    b2 = b.reshape(1, _NE).astype(jnp.float32)

    grid = (n // _BM,)
    return pl.pallas_call(
        _router_body,
        grid=grid,
        in_specs=[
            pl.BlockSpec((_BM, _K1), lambda m: (m, 0)),
            pl.BlockSpec((_BM, _K2), lambda m: (m, 1)),
            pl.BlockSpec((_BM, _K3), lambda m: (m, (_K1 + _K2) // _K3)),
            pl.BlockSpec((_K1, _NE), lambda m: (0, 0)),
            pl.BlockSpec((_K2, _NE), lambda m: (0, 0)),
            pl.BlockSpec((_K3, _NE), lambda m: (0, 0)),
            pl.BlockSpec((1, _NE), lambda m: (0, 0)),
        ],
        out_specs=pl.BlockSpec((_BM, _NE), lambda m: (m, 0)),
        out_shape=jax.ShapeDtypeStruct((n, _NE), jnp.float32),
        compiler_params=pltpu.CompilerParams(
            dimension_semantics=("parallel",),
        ),
    )(x, x, x, w1, w2, w3, b2)


# speedup vs baseline: 1.1865x; 1.1865x over previous
import jax
import jax.numpy as jnp
from jax.experimental import pallas as pl
from jax.experimental.pallas import tpu as pltpu

_WIDTH = 2048
_KPAD = 2176
_NE = 64
_BM = 2048


def _router_body(x_ref, w_ref, b_ref, o_ref):
    acc = jax.lax.dot_general(
        w_ref[...], x_ref[...],
        dimension_numbers=(((1,), (1,)), ((), ())),
        preferred_element_type=jnp.float32,
    )
    o_ref[...] = acc + b_ref[...]


def kernel(x, A, W, b):
    n = x.shape[0]
    off = (A - _WIDTH).astype(jnp.int32) if hasattr(A, "astype") else jnp.int32(A - _WIDTH)
    wp = jax.lax.dynamic_update_slice(
        jnp.zeros((_NE, _KPAD), jnp.float32), W.astype(jnp.float32), (0, off)
    )
    b2 = b.reshape(_NE, 1).astype(jnp.float32)

    grid = (n // _BM,)
    out_t = pl.pallas_call(
        _router_body,
        grid=grid,
        in_specs=[
            pl.BlockSpec((_BM, _KPAD), lambda m: (m, 0)),
            pl.BlockSpec((_NE, _KPAD), lambda m: (0, 0)),
            pl.BlockSpec((_NE, 1), lambda m: (0, 0)),
        ],
        out_specs=pl.BlockSpec((_NE, _BM), lambda m: (0, m)),
        out_shape=jax.ShapeDtypeStruct((_NE, n), jnp.float32),
        compiler_params=pltpu.CompilerParams(
            dimension_semantics=("parallel",),
        ),
    )(x, wp, b2)
    return out_t.T


# speedup vs baseline: 1.4097x; 1.1881x over previous
import jax
import jax.numpy as jnp
from jax.experimental import pallas as pl
from jax.experimental.pallas import tpu as pltpu

_WIDTH = 2048
_KPAD = 2176
_NE = 64
_BM = 1024


def _router_body(off_ref, x_ref, w_ref, b_ref, o_ref):
    wfull = jnp.concatenate(
        [w_ref[...], jnp.zeros((_NE, _KPAD - _WIDTH), jnp.float32)], axis=1
    )
    wp = pltpu.roll(wfull, off_ref[0], axis=1)
    acc = jax.lax.dot_general(
        wp, x_ref[...],
        dimension_numbers=(((1,), (1,)), ((), ())),
        preferred_element_type=jnp.float32,
    )
    o_ref[...] = acc + b_ref[:, 0:1]


def kernel(x, A, W, b):
    n = x.shape[0]
    a32 = A.astype(jnp.int32) if hasattr(A, "astype") else jnp.int32(A)
    off = jnp.reshape(a32 - _WIDTH, (1,))
    b2 = jnp.broadcast_to(b.reshape(_NE, 1).astype(jnp.float32), (_NE, 128))

    out_t = pl.pallas_call(
        _router_body,
        grid_spec=pltpu.PrefetchScalarGridSpec(
            num_scalar_prefetch=1,
            grid=(n // _BM,),
            in_specs=[
                pl.BlockSpec((_BM, _KPAD), lambda m, off_ref: (m, 0)),
                pl.BlockSpec((_NE, _WIDTH), lambda m, off_ref: (0, 0)),
                pl.BlockSpec((_NE, 128), lambda m, off_ref: (0, 0)),
            ],
            out_specs=pl.BlockSpec((_NE, _BM), lambda m, off_ref: (0, m)),
        ),
        out_shape=jax.ShapeDtypeStruct((_NE, n), jnp.float32),
        compiler_params=pltpu.CompilerParams(
            dimension_semantics=("parallel",),
        ),
    )(off, x, W, b2)
    return out_t.T


# speedup vs baseline: 1.4913x; 1.0579x over previous
import jax
import jax.numpy as jnp
from jax.experimental import pallas as pl
from jax.experimental.pallas import tpu as pltpu

_WIDTH = 2048
_KPAD = 2176
_NE = 64
_BM = 1024


def _router_body(off_ref, x_ref, w_ref, b_ref, o_ref):
    wfull = jnp.concatenate(
        [w_ref[...], jnp.zeros((_NE, _KPAD - _WIDTH), jnp.float32)], axis=1
    )
    wp = pltpu.roll(wfull, off_ref[0], axis=1)
    acc = jax.lax.dot_general(
        wp, x_ref[...],
        dimension_numbers=(((1,), (1,)), ((), ())),
        preferred_element_type=jnp.float32,
    )
    rows = jax.lax.broadcasted_iota(jnp.int32, (_NE, _NE), 0)
    cols = jax.lax.broadcasted_iota(jnp.int32, (_NE, _NE), 1)
    eye = jnp.where(rows == cols, 1.0, 0.0).astype(jnp.float32)
    b_col = jax.lax.dot_general(
        eye, b_ref[...],
        dimension_numbers=(((1,), (1,)), ((), ())),
        preferred_element_type=jnp.float32,
    )
    o_ref[...] = acc + b_col


def kernel(x, A, W, b):
    n = x.shape[0]
    a32 = A.astype(jnp.int32) if hasattr(A, "astype") else jnp.int32(A)
    off = jnp.reshape(a32 - _WIDTH, (1,))
    b2 = b.reshape(1, _NE)

    out_t = pl.pallas_call(
        _router_body,
        grid_spec=pltpu.PrefetchScalarGridSpec(
            num_scalar_prefetch=1,
            grid=(n // _BM,),
            in_specs=[
                pl.BlockSpec((_BM, _KPAD), lambda m, off_ref: (m, 0)),
                pl.BlockSpec((_NE, _WIDTH), lambda m, off_ref: (0, 0)),
                pl.BlockSpec((1, _NE), lambda m, off_ref: (0, 0)),
            ],
            out_specs=pl.BlockSpec((_NE, _BM), lambda m, off_ref: (0, m)),
        ),
        out_shape=jax.ShapeDtypeStruct((_NE, n), jnp.float32),
        compiler_params=pltpu.CompilerParams(
            dimension_semantics=("parallel",),
        ),
    )(off, x, W, b2)
    return out_t.T
